# baseline (speedup 1.0000x reference)
import functools

import jax
import jax.numpy as jnp
from jax import lax
from jax.experimental import pallas as pl
from jax.experimental.pallas import tpu as pltpu
from jax.experimental.pallas import tpu_sc as plsc

_N = 50000
_E = 800000
_H = 64
_NG = 512
_NS = 64
_NC = 32

_NSUB = 16
_EPT0 = _E // (2 * _NSUB)
_CH = 336
_PAD = 200
_EPT2 = _EPT0 + _PAD
_NCH2 = _EPT2 // _CH
_RPT = 3136
_NPAD = _NSUB * _RPT
_TRASH = _NPAD - 1

_BN = 1000
_GRID = _N // _BN


def _edge_body(xb_hbm, srcp_hbm, dstp_hbm, zdeg_hbm, zrow_hbm, ones_hbm,
               agg_hbm, deg_hbm,
               src0, src1, dst0, dst1, dst2, rows0, rows1, onesv,
               sg0, sg1, ss0, ss1, sd0, sd1, si0, si1,
               agg_s, deg_s):
  c = lax.axis_index("c")
  s = lax.axis_index("s")
  srcb = (src0, src1)
  dstb = (dst0, dst1, dst2)
  rowsb = (rows0, rows1)
  sg = (sg0, sg1)
  ss = (ss0, ss1)
  sd = (sd0, sd1)
  si = (si0, si1)

  pltpu.sync_copy(zrow_hbm, rows0)
  pltpu.sync_copy(ones_hbm, onesv)
  for jj in range(_RPT // _CH):
    pltpu.sync_copy(rows0, agg_s.at[pl.ds(s * _RPT + jj * _CH, _CH)])
  _TAIL = _RPT - (_RPT // _CH) * _CH
  pltpu.sync_copy(rows0.at[pl.ds(0, _TAIL)],
                  agg_s.at[pl.ds(s * _RPT + (_RPT // _CH) * _CH, _TAIL)])
  pltpu.sync_copy(zdeg_hbm, deg_s.at[pl.ds(s * _RPT, _RPT)])

  plsc.subcore_barrier()

  base = (c * _NSUB + s) * _EPT2

  def _fire_idx(k, sb, db, sem):
    pltpu.async_copy(srcp_hbm.at[pl.ds(base + k * _CH, _CH)], sb, sem)
    pltpu.async_copy(dstp_hbm.at[pl.ds(base + k * _CH, _CH)], db, sem)

  def _wait_idx(k, sb, db, sem):
    pltpu.make_async_copy(srcp_hbm.at[pl.ds(base + k * _CH, _CH)], sb,
                          sem).wait()
    pltpu.make_async_copy(dstp_hbm.at[pl.ds(base + k * _CH, _CH)], db,
                          sem).wait()

  _fire_idx(0, srcb[0], dstb[0], si[0])
  _fire_idx(1, srcb[1], dstb[1], si[1])
  _wait_idx(0, srcb[0], dstb[0], si[0])
  pltpu.async_copy(xb_hbm.at[srcb[0]], rowsb[0], sg[0])

  def _outer(i, carry):
    for j in range(6):
      k = 6 * i + j
      p = j % 2
      q = 1 - p
      m = j % 3

      @pl.when(k < _NCH2)
      def _body():
        @pl.when(k >= 1)
        def _wait_prev():
          pltpu.make_async_copy(rowsb[q], agg_s.at[dstb[(j - 1) % 3]],
                                ss[q]).wait()
          pltpu.make_async_copy(onesv, deg_s.at[dstb[(j - 1) % 3]],
                                sd[q]).wait()

        pltpu.make_async_copy(xb_hbm.at[srcb[p]], rowsb[p], sg[p]).wait()
        pltpu.async_copy(rowsb[p], agg_s.at[dstb[m]], ss[p], add=True)
        pltpu.async_copy(onesv, deg_s.at[dstb[m]], sd[p], add=True)

        @pl.when(k + 2 < _NCH2)
        def _prefetch():
          _fire_idx(k + 2, srcb[p], dstb[(j + 2) % 3], si[j % 2])

        @pl.when(k + 1 < _NCH2)
        def _next_gather():
          _wait_idx(k + 1, srcb[q], dstb[(j + 1) % 3], si[(j + 1) % 2])
          pltpu.async_copy(xb_hbm.at[srcb[q]], rowsb[q], sg[q])
    return carry
  lax.fori_loop(0, (_NCH2 + 5) // 6, _outer, 0)

  _PL = (_NCH2 - 1) % 2
  _ML = (_NCH2 - 1) % 3
  pltpu.make_async_copy(rowsb[_PL], agg_s.at[dstb[_ML]], ss[_PL]).wait()
  pltpu.make_async_copy(onesv, deg_s.at[dstb[_ML]], sd[_PL]).wait()

  plsc.subcore_barrier()

  pltpu.sync_copy(agg_s.at[pl.ds(s * _RPT, _RPT)],
                  agg_hbm.at[c, pl.ds(s * _RPT, _RPT)])
  pltpu.sync_copy(deg_s.at[pl.ds(s * _RPT, _RPT)],
                  deg_hbm.at[c, pl.ds(s * _RPT, _RPT)])


_edge_pass = pl.kernel(
    _edge_body,
    out_type=[
        jax.ShapeDtypeStruct((2, _NPAD, _H), jnp.bfloat16),
        jax.ShapeDtypeStruct((2, _NPAD), jnp.float32),
    ],
    mesh=plsc.VectorSubcoreMesh(core_axis_name="c", subcore_axis_name="s"),
    compiler_params=pltpu.CompilerParams(use_tc_tiling_on_sc=False),
    scratch_types=[
        pltpu.VMEM((_CH,), jnp.int32),
        pltpu.VMEM((_CH,), jnp.int32),
        pltpu.VMEM((_CH,), jnp.int32),
        pltpu.VMEM((_CH,), jnp.int32),
        pltpu.VMEM((_CH,), jnp.int32),
        pltpu.VMEM((_CH, _H), jnp.bfloat16),
        pltpu.VMEM((_CH, _H), jnp.bfloat16),
        pltpu.VMEM((_CH,), jnp.float32),
        pltpu.SemaphoreType.DMA,
        pltpu.SemaphoreType.DMA,
        pltpu.SemaphoreType.DMA,
        pltpu.SemaphoreType.DMA,
        pltpu.SemaphoreType.DMA,
        pltpu.SemaphoreType.DMA,
        pltpu.SemaphoreType.DMA,
        pltpu.SemaphoreType.DMA,
        pltpu.VMEM_SHARED((_NPAD, _H), jnp.bfloat16),
        pltpu.VMEM_SHARED((_NPAD,), jnp.float32),
    ],
)


def _embed_body(sid_ref, cid_ref, semb_ref, cemb_ref, x_ref, xb_ref):
  ohs = (sid_ref[...] == lax.broadcasted_iota(jnp.int32, (_BN, _NS), 1))
  ohc = (cid_ref[...] == lax.broadcasted_iota(jnp.int32, (_BN, _NC), 1))
  v = (jnp.dot(ohs.astype(jnp.float32), semb_ref[...],
               preferred_element_type=jnp.float32)
       + jnp.dot(ohc.astype(jnp.float32), cemb_ref[...],
                 preferred_element_type=jnp.float32))
  x_ref[...] = v
  xb_ref[...] = v.astype(jnp.bfloat16)


def _embed(shape_id, colour_id, shape_emb, col_emb):
  return pl.pallas_call(
      _embed_body,
      grid=(_GRID,),
      in_specs=[
          pl.BlockSpec((_BN, 1), lambda i: (i, 0)),
          pl.BlockSpec((_BN, 1), lambda i: (i, 0)),
          pl.BlockSpec((_NS, _H), lambda i: (0, 0)),
          pl.BlockSpec((_NC, _H), lambda i: (0, 0)),
      ],
      out_specs=[
          pl.BlockSpec((_BN, _H), lambda i: (i, 0)),
          pl.BlockSpec((_BN, _H), lambda i: (i, 0)),
      ],
      out_shape=[
          jax.ShapeDtypeStruct((_N, _H), jnp.float32),
          jax.ShapeDtypeStruct((_N, _H), jnp.bfloat16),
      ],
  )(shape_id.reshape(_N, 1), colour_id.reshape(_N, 1), shape_emb, col_emb)


def _mm_body(aggh_ref, deg_ref, x_ref, wc_ref, b_ref, h_ref, st_ref):
  d = deg_ref[0] + deg_ref[1]
  dinv = 1.0 / jnp.maximum(d, 1.0)
  a = (aggh_ref[0].astype(jnp.float32)
       + aggh_ref[1].astype(jnp.float32)) * dinv
  cat = jnp.concatenate([x_ref[...], a], axis=1)
  h = jnp.dot(cat, wc_ref[...], preferred_element_type=jnp.float32) + b_ref[...]
  h_ref[...] = h
  s1 = jnp.sum(h, axis=0, keepdims=True)
  s2 = jnp.sum(h * h, axis=0, keepdims=True)
  st = jnp.concatenate(
      [s1, s2, jnp.zeros((6, _H), jnp.float32)], axis=0)

  @pl.when(pl.program_id(0) == 0)
  def _():
    st_ref[...] = st

  @pl.when(pl.program_id(0) > 0)
  def _():
    st_ref[...] += st


def _mm(aggh, deg, x, wc, b):
  return pl.pallas_call(
      _mm_body,
      grid=(_GRID,),
      in_specs=[
          pl.BlockSpec((2, _BN, _H), lambda i: (0, i, 0)),
          pl.BlockSpec((2, _BN, 1), lambda i: (0, i, 0)),
          pl.BlockSpec((_BN, _H), lambda i: (i, 0)),
          pl.BlockSpec((2 * _H, _H), lambda i: (0, 0)),
          pl.BlockSpec((1, _H), lambda i: (0, 0)),
      ],
      out_specs=[
          pl.BlockSpec((_BN, _H), lambda i: (i, 0)),
          pl.BlockSpec((8, _H), lambda i: (0, 0)),
      ],
      out_shape=[
          jax.ShapeDtypeStruct((_N, _H), jnp.float32),
          jax.ShapeDtypeStruct((8, _H), jnp.float32),
      ],
  )(aggh, deg, x, wc, b.reshape(1, _H))


def _norm_body(h_ref, st_ref, g_ref, b_ref, o_ref, ob_ref):
  st = st_ref[...]
  m = st[0:1] * (1.0 / _N)
  v = st[1:2] * (1.0 / _N) - m * m
  inv = lax.rsqrt(v + 1e-5)
  o = jnp.maximum((h_ref[...] - m) * inv * g_ref[...] + b_ref[...], 0.0)
  o_ref[...] = o
  ob_ref[...] = o.astype(jnp.bfloat16)


def _norm(h, st, g, b):
  return pl.pallas_call(
      _norm_body,
      grid=(_GRID,),
      in_specs=[
          pl.BlockSpec((_BN, _H), lambda i: (i, 0)),
          pl.BlockSpec((8, _H), lambda i: (0, 0)),
          pl.BlockSpec((1, _H), lambda i: (0, 0)),
          pl.BlockSpec((1, _H), lambda i: (0, 0)),
      ],
      out_specs=[
          pl.BlockSpec((_BN, _H), lambda i: (i, 0)),
          pl.BlockSpec((_BN, _H), lambda i: (i, 0)),
      ],
      out_shape=[
          jax.ShapeDtypeStruct((_N, _H), jnp.float32),
          jax.ShapeDtypeStruct((_N, _H), jnp.bfloat16),
      ],
  )(h, st, g.reshape(1, _H), b.reshape(1, _H))


def _final_body(h_ref, st_ref, g_ref, b_ref, batch_ref, w_ref, bias_ref, o_ref):
  st = st_ref[...]
  m = st[0:1] * (1.0 / _N)
  v = st[1:2] * (1.0 / _N) - m * m
  inv = lax.rsqrt(v + 1e-5)
  xb = jnp.maximum(
      (h_ref[...] - m) * inv * g_ref[...] + b_ref[...], 0.0)
  oh = (batch_ref[...] == lax.broadcasted_iota(jnp.int32, (_BN, _NG), 1))
  gxp = lax.dot_general(oh.astype(jnp.float32), xb,
                        (((0,), (0,)), ((), ())),
                        preferred_element_type=jnp.float32)
  op = jnp.dot(gxp, w_ref[...], preferred_element_type=jnp.float32)

  @pl.when(pl.program_id(0) == 0)
  def _():
    o_ref[...] = op + bias_ref[...]

  @pl.when(pl.program_id(0) > 0)
  def _():
    o_ref[...] += op


def _final(h, st, g, b, batch, w_pad, bias_pad):
  return pl.pallas_call(
      _final_body,
      grid=(_GRID,),
      in_specs=[
          pl.BlockSpec((_BN, _H), lambda i: (i, 0)),
          pl.BlockSpec((8, _H), lambda i: (0, 0)),
          pl.BlockSpec((1, _H), lambda i: (0, 0)),
          pl.BlockSpec((1, _H), lambda i: (0, 0)),
          pl.BlockSpec((_BN, 1), lambda i: (i, 0)),
          pl.BlockSpec((_H, 128), lambda i: (0, 0)),
          pl.BlockSpec((1, 128), lambda i: (0, 0)),
      ],
      out_specs=pl.BlockSpec((_NG, 128), lambda i: (0, 0)),
      out_shape=jax.ShapeDtypeStruct((_NG, 128), jnp.float32),
  )(h, st, g.reshape(1, _H), b.reshape(1, _H), batch.reshape(_N, 1),
    w_pad, bias_pad)


def kernel(shape_id, colour_id, edge_index, batch, shape_emb, col_emb,
           W1l, b1l, W1r, bn1_g, bn1_b, W2l, b2l, W2r, bn2_g, bn2_b,
           lin_W, lin_b):
  src = edge_index[0]
  dst = edge_index[1]
  wc1 = jnp.concatenate([W1r, W1l], axis=0)
  wc2 = jnp.concatenate([W2r, W2l], axis=0)
  w_pad = jnp.pad(lin_W, ((0, 0), (0, 128 - lin_W.shape[1])))
  bias_pad = jnp.pad(lin_b, (0, 128 - lin_b.shape[0])).reshape(1, 128)

  srcp = jnp.pad(src.reshape(2 * _NSUB, _EPT0),
                 ((0, 0), (0, _PAD))).reshape(-1)
  dstp = jnp.pad(dst.reshape(2 * _NSUB, _EPT0), ((0, 0), (0, _PAD)),
                 constant_values=_TRASH).reshape(-1)

  zdeg = jnp.zeros((_RPT,), jnp.float32)
  zrow = jnp.zeros((_CH, _H), jnp.bfloat16)
  ones16 = jnp.ones((_CH,), jnp.float32)

  x, xb = _embed(shape_id, colour_id, shape_emb, col_emb)
  agg1, deg = _edge_pass(xb, srcp, dstp, zdeg, zrow, ones16)
  degf = deg.reshape(2, _NPAD, 1)
  h1, st1 = _mm(agg1, degf, x, wc1, b1l)
  x1, x1b = _norm(h1, st1, bn1_g, bn1_b)
  agg2, _ = _edge_pass(x1b, srcp, dstp, zdeg, zrow, ones16)
  h2, st2 = _mm(agg2, degf, x1, wc2, b2l)
  out = _final(h2, st2, bn2_g, bn2_b, batch, w_pad, bias_pad)
  return out[:, : lin_b.shape[0]]


# speedup vs baseline: 1.3024x; 1.3024x over previous
import functools

import jax
import jax.numpy as jnp
from jax import lax
from jax.experimental import pallas as pl
from jax.experimental.pallas import tpu as pltpu
from jax.experimental.pallas import tpu_sc as plsc

_N = 50000
_E = 800000
_H = 64
_NG = 512
_NS = 64
_NC = 32

_NSUB = 16
_EPT = _E // (2 * _NSUB)
_CH = 304
_NF = _EPT // _CH
_CT = _EPT - _NF * _CH
_RPT = 3136
_NPAD = _NSUB * _RPT

_BN = 10000
_GRID = _N // _BN
_BNF = 5000
_GRIDF = _N // _BNF


def _edge_body(xb_hbm, ei_hbm, zdeg_hbm, zrow_hbm, ones_hbm,
               agg_hbm, deg_hbm,
               src0, src1, dst0, dst1, dst2, rows0, rows1, onesv,
               tsrc, tdst, trows,
               sg0, sg1, ss0, ss1, sd0, sd1, si0, si1,
               agg_s, deg_s):
  c = lax.axis_index("c")
  s = lax.axis_index("s")
  srcb = (src0, src1)
  dstb = (dst0, dst1, dst2)
  rowsb = (rows0, rows1)
  sg = (sg0, sg1)
  ss = (ss0, ss1)
  sd = (sd0, sd1)
  si = (si0, si1)

  pltpu.sync_copy(zrow_hbm, rows0)
  pltpu.sync_copy(ones_hbm, onesv)
  for jj in range(_RPT // _CH):
    pltpu.sync_copy(rows0, agg_s.at[pl.ds(s * _RPT + jj * _CH, _CH)])
  _TAIL = _RPT - (_RPT // _CH) * _CH
  pltpu.sync_copy(rows0.at[pl.ds(0, _TAIL)],
                  agg_s.at[pl.ds(s * _RPT + (_RPT // _CH) * _CH, _TAIL)])
  pltpu.sync_copy(zdeg_hbm, deg_s.at[pl.ds(s * _RPT, _RPT)])

  plsc.subcore_barrier()

  base = (c * _NSUB + s) * _EPT

  def _fire_idx(k, sb, db, sem):
    pltpu.async_copy(ei_hbm.at[0, pl.ds(base + k * _CH, _CH)], sb, sem)
    pltpu.async_copy(ei_hbm.at[1, pl.ds(base + k * _CH, _CH)], db, sem)

  def _wait_idx(k, sb, db, sem):
    pltpu.make_async_copy(ei_hbm.at[0, pl.ds(base + k * _CH, _CH)], sb,
                          sem).wait()
    pltpu.make_async_copy(ei_hbm.at[1, pl.ds(base + k * _CH, _CH)], db,
                          sem).wait()

  _fire_idx(0, srcb[0], dstb[0], si[0])
  _fire_idx(1, srcb[1], dstb[1], si[1])
  _wait_idx(0, srcb[0], dstb[0], si[0])
  pltpu.async_copy(xb_hbm.at[srcb[0]], rowsb[0], sg[0])

  def _outer(i, carry):
    for j in range(6):
      k = 6 * i + j
      p = j % 2
      q = 1 - p
      m = j % 3

      @pl.when(k < _NF)
      def _body():
        @pl.when(k >= 1)
        def _wait_prev():
          pltpu.make_async_copy(rowsb[q], agg_s.at[dstb[(j - 1) % 3]],
                                ss[q]).wait()
          pltpu.make_async_copy(onesv, deg_s.at[dstb[(j - 1) % 3]],
                                sd[q]).wait()

        pltpu.make_async_copy(xb_hbm.at[srcb[p]], rowsb[p], sg[p]).wait()
        pltpu.async_copy(rowsb[p], agg_s.at[dstb[m]], ss[p], add=True)
        pltpu.async_copy(onesv, deg_s.at[dstb[m]], sd[p], add=True)

        @pl.when(k + 2 < _NF)
        def _prefetch():
          _fire_idx(k + 2, srcb[p], dstb[(j + 2) % 3], si[j % 2])

        @pl.when(k + 1 < _NF)
        def _next_gather():
          _wait_idx(k + 1, srcb[q], dstb[(j + 1) % 3], si[(j + 1) % 2])
          pltpu.async_copy(xb_hbm.at[srcb[q]], rowsb[q], sg[q])
    return carry
  lax.fori_loop(0, (_NF + 5) // 6, _outer, 0)

  _PL = (_NF - 1) % 2
  _ML = (_NF - 1) % 3
  pltpu.make_async_copy(rowsb[_PL], agg_s.at[dstb[_ML]], ss[_PL]).wait()
  pltpu.make_async_copy(onesv, deg_s.at[dstb[_ML]], sd[_PL]).wait()

  tb = base + _NF * _CH
  pltpu.sync_copy(ei_hbm.at[0, pl.ds(tb, _CT)], tsrc)
  pltpu.sync_copy(ei_hbm.at[1, pl.ds(tb, _CT)], tdst)
  pltpu.async_copy(xb_hbm.at[tsrc], trows, sg0)
  pltpu.make_async_copy(xb_hbm.at[tsrc], trows, sg0).wait()
  pltpu.async_copy(trows, agg_s.at[tdst], ss0, add=True)
  pltpu.async_copy(onesv.at[pl.ds(0, _CT)], deg_s.at[tdst], sd0, add=True)
  pltpu.make_async_copy(trows, agg_s.at[tdst], ss0).wait()
  pltpu.make_async_copy(onesv.at[pl.ds(0, _CT)], deg_s.at[tdst], sd0).wait()

  plsc.subcore_barrier()

  pltpu.sync_copy(agg_s.at[pl.ds(s * _RPT, _RPT)],
                  agg_hbm.at[c, pl.ds(s * _RPT, _RPT)])
  pltpu.sync_copy(deg_s.at[pl.ds(s * _RPT, _RPT)],
                  deg_hbm.at[c, pl.ds(s * _RPT, _RPT)])


_edge_pass = pl.kernel(
    _edge_body,
    out_type=[
        jax.ShapeDtypeStruct((2, _NPAD, _H), jnp.bfloat16),
        jax.ShapeDtypeStruct((2, _NPAD), jnp.float32),
    ],
    mesh=plsc.VectorSubcoreMesh(core_axis_name="c", subcore_axis_name="s"),
    compiler_params=pltpu.CompilerParams(use_tc_tiling_on_sc=False),
    scratch_types=[
        pltpu.VMEM((_CH,), jnp.int32),
        pltpu.VMEM((_CH,), jnp.int32),
        pltpu.VMEM((_CH,), jnp.int32),
        pltpu.VMEM((_CH,), jnp.int32),
        pltpu.VMEM((_CH,), jnp.int32),
        pltpu.VMEM((_CH, _H), jnp.bfloat16),
        pltpu.VMEM((_CH, _H), jnp.bfloat16),
        pltpu.VMEM((_CH,), jnp.float32),
        pltpu.VMEM((_CT,), jnp.int32),
        pltpu.VMEM((_CT,), jnp.int32),
        pltpu.VMEM((_CT, _H), jnp.bfloat16),
        pltpu.SemaphoreType.DMA,
        pltpu.SemaphoreType.DMA,
        pltpu.SemaphoreType.DMA,
        pltpu.SemaphoreType.DMA,
        pltpu.SemaphoreType.DMA,
        pltpu.SemaphoreType.DMA,
        pltpu.SemaphoreType.DMA,
        pltpu.SemaphoreType.DMA,
        pltpu.VMEM_SHARED((_NPAD, _H), jnp.bfloat16),
        pltpu.VMEM_SHARED((_NPAD,), jnp.float32),
    ],
)


def _embed_body(sid_ref, cid_ref, semb_ref, cemb_ref, x_ref):
  ohs = (sid_ref[...] == lax.broadcasted_iota(jnp.int32, (_BN, _NS), 1))
  ohc = (cid_ref[...] == lax.broadcasted_iota(jnp.int32, (_BN, _NC), 1))
  x_ref[...] = (
      jnp.dot(ohs.astype(jnp.float32), semb_ref[...],
              preferred_element_type=jnp.float32)
      + jnp.dot(ohc.astype(jnp.float32), cemb_ref[...],
                preferred_element_type=jnp.float32))


def _embed(shape_id, colour_id, shape_emb, col_emb):
  return pl.pallas_call(
      _embed_body,
      grid=(_GRID,),
      in_specs=[
          pl.BlockSpec((_BN, 1), lambda i: (i, 0)),
          pl.BlockSpec((_BN, 1), lambda i: (i, 0)),
          pl.BlockSpec((_NS, _H), lambda i: (0, 0)),
          pl.BlockSpec((_NC, _H), lambda i: (0, 0)),
      ],
      out_specs=pl.BlockSpec((_BN, _H), lambda i: (i, 0)),
      out_shape=jax.ShapeDtypeStruct((_N, _H), jnp.float32),
  )(shape_id.reshape(_N, 1), colour_id.reshape(_N, 1), shape_emb, col_emb)


def _mm_body(aggh_ref, deg_ref, x_ref, wc_ref, b_ref, h_ref, st_ref):
  d = deg_ref[0] + deg_ref[1]
  dinv = 1.0 / jnp.maximum(d, 1.0)
  a = (aggh_ref[0].astype(jnp.float32)
       + aggh_ref[1].astype(jnp.float32)) * dinv
  cat = jnp.concatenate([x_ref[...], a], axis=1)
  h = jnp.dot(cat, wc_ref[...], preferred_element_type=jnp.float32) + b_ref[...]
  h_ref[...] = h
  s1 = jnp.sum(h, axis=0, keepdims=True)
  s2 = jnp.sum(h * h, axis=0, keepdims=True)
  st = jnp.concatenate(
      [s1, s2, jnp.zeros((6, _H), jnp.float32)], axis=0)

  @pl.when(pl.program_id(0) == 0)
  def _():
    st_ref[...] = st

  @pl.when(pl.program_id(0) > 0)
  def _():
    st_ref[...] += st


def _mm(aggh, deg, x, wc, b):
  return pl.pallas_call(
      _mm_body,
      grid=(_GRID,),
      in_specs=[
          pl.BlockSpec((2, _BN, _H), lambda i: (0, i, 0)),
          pl.BlockSpec((2, _BN, 1), lambda i: (0, i, 0)),
          pl.BlockSpec((_BN, _H), lambda i: (i, 0)),
          pl.BlockSpec((2 * _H, _H), lambda i: (0, 0)),
          pl.BlockSpec((1, _H), lambda i: (0, 0)),
      ],
      out_specs=[
          pl.BlockSpec((_BN, _H), lambda i: (i, 0)),
          pl.BlockSpec((8, _H), lambda i: (0, 0)),
      ],
      out_shape=[
          jax.ShapeDtypeStruct((_N, _H), jnp.float32),
          jax.ShapeDtypeStruct((8, _H), jnp.float32),
      ],
  )(aggh, deg, x, wc, b.reshape(1, _H))


def _norm_body(h_ref, st_ref, g_ref, b_ref, o_ref):
  st = st_ref[...]
  m = st[0:1] * (1.0 / _N)
  v = st[1:2] * (1.0 / _N) - m * m
  inv = lax.rsqrt(v + 1e-5)
  o_ref[...] = jnp.maximum(
      (h_ref[...] - m) * inv * g_ref[...] + b_ref[...], 0.0)


def _norm(h, st, g, b):
  return pl.pallas_call(
      _norm_body,
      grid=(_GRID,),
      in_specs=[
          pl.BlockSpec((_BN, _H), lambda i: (i, 0)),
          pl.BlockSpec((8, _H), lambda i: (0, 0)),
          pl.BlockSpec((1, _H), lambda i: (0, 0)),
          pl.BlockSpec((1, _H), lambda i: (0, 0)),
      ],
      out_specs=pl.BlockSpec((_BN, _H), lambda i: (i, 0)),
      out_shape=jax.ShapeDtypeStruct((_N, _H), jnp.float32),
  )(h, st, g.reshape(1, _H), b.reshape(1, _H))


def _final_body(h_ref, st_ref, g_ref, b_ref, batch_ref, w_ref, bias_ref, o_ref):
  st = st_ref[...]
  m = st[0:1] * (1.0 / _N)
  v = st[1:2] * (1.0 / _N) - m * m
  inv = lax.rsqrt(v + 1e-5)
  xb = jnp.maximum(
      (h_ref[...] - m) * inv * g_ref[...] + b_ref[...], 0.0)
  oh = (batch_ref[...] == lax.broadcasted_iota(jnp.int32, (_BNF, _NG), 1))
  gxp = lax.dot_general(oh.astype(jnp.float32), xb,
                        (((0,), (0,)), ((), ())),
                        preferred_element_type=jnp.float32)
  op = jnp.dot(gxp, w_ref[...], preferred_element_type=jnp.float32)

  @pl.when(pl.program_id(0) == 0)
  def _():
    o_ref[...] = op + bias_ref[...]

  @pl.when(pl.program_id(0) > 0)
  def _():
    o_ref[...] += op


def _final(h, st, g, b, batch, w_pad, bias_pad):
  return pl.pallas_call(
      _final_body,
      grid=(_GRIDF,),
      in_specs=[
          pl.BlockSpec((_BNF, _H), lambda i: (i, 0)),
          pl.BlockSpec((8, _H), lambda i: (0, 0)),
          pl.BlockSpec((1, _H), lambda i: (0, 0)),
          pl.BlockSpec((1, _H), lambda i: (0, 0)),
          pl.BlockSpec((_BNF, 1), lambda i: (i, 0)),
          pl.BlockSpec((_H, 128), lambda i: (0, 0)),
          pl.BlockSpec((1, 128), lambda i: (0, 0)),
      ],
      out_specs=pl.BlockSpec((_NG, 128), lambda i: (0, 0)),
      out_shape=jax.ShapeDtypeStruct((_NG, 128), jnp.float32),
  )(h, st, g.reshape(1, _H), b.reshape(1, _H), batch.reshape(_N, 1),
    w_pad, bias_pad)


def kernel(shape_id, colour_id, edge_index, batch, shape_emb, col_emb,
           W1l, b1l, W1r, bn1_g, bn1_b, W2l, b2l, W2r, bn2_g, bn2_b,
           lin_W, lin_b):
  wc1 = jnp.concatenate([W1r, W1l], axis=0)
  wc2 = jnp.concatenate([W2r, W2l], axis=0)
  w_pad = jnp.pad(lin_W, ((0, 0), (0, 128 - lin_W.shape[1])))
  bias_pad = jnp.pad(lin_b, (0, 128 - lin_b.shape[0])).reshape(1, 128)

  zdeg = jnp.zeros((_RPT,), jnp.float32)
  zrow = jnp.zeros((_CH, _H), jnp.bfloat16)
  onesf = jnp.ones((_CH,), jnp.float32)

  x = _embed(shape_id, colour_id, shape_emb, col_emb)
  xb = x.astype(jnp.bfloat16)
  agg1, deg = _edge_pass(xb, edge_index, zdeg, zrow, onesf)
  degf = deg.reshape(2, _NPAD, 1)
  h1, st1 = _mm(agg1, degf, x, wc1, b1l)
  x1 = _norm(h1, st1, bn1_g, bn1_b)
  x1b = x1.astype(jnp.bfloat16)
  agg2, _ = _edge_pass(x1b, edge_index, zdeg, zrow, onesf)
  h2, st2 = _mm(agg2, degf, x1, wc2, b2l)
  out = _final(h2, st2, bn2_g, bn2_b, batch, w_pad, bias_pad)
  return out[:, : lin_b.shape[0]]
